# Initial kernel scaffold; baseline (speedup 1.0000x reference)
import jax
import jax.numpy as jnp
from jax.experimental import pallas as pl


def kernel(x, tables):
    raise NotImplementedError("write your pallas kernel here")


# speedup vs baseline: 1.2023x; 1.2023x over previous
import functools

import jax
import jax.numpy as jnp
from jax import lax
from jax.experimental import pallas as pl
from jax.experimental.pallas import tpu as pltpu
from jax.experimental.pallas import tpu_sc as plsc

_FIELDS = 26
_VOCAB = 100000
_EMB = 32
_LANES = 16


def _body(n_chunk, chunk, rows_per_w, nc, x_hbm, tab_hbm, out_hbm,
          idx_v, rows_v, sem):
    wid = lax.axis_index("s") * nc + lax.axis_index("c")
    lane = lax.iota(jnp.int32, _LANES)

    def do_chunk(c, _):
        base = wid * rows_per_w + c * chunk
        pltpu.sync_copy(x_hbm.at[pl.ds(base, chunk)], idx_v)

        def add_offs(j, _):
            r = base + j * _LANES + lane
            f = lax.rem(r, _FIELDS)
            sl = pl.ds(j * _LANES, _LANES)
            idx_v[sl] = idx_v[sl] + f * _VOCAB
            return 0

        lax.fori_loop(0, chunk // _LANES, add_offs, 0, unroll=4)
        pltpu.async_copy(tab_hbm.at[idx_v], rows_v, sem).wait()
        pltpu.sync_copy(rows_v, out_hbm.at[pl.ds(base, chunk)])
        return 0

    lax.fori_loop(0, n_chunk, do_chunk, 0)


def kernel(x, tables):
    batch = x.shape[0]
    rows = batch * _FIELDS
    x_flat = x.reshape(rows)
    tab_flat = tables.reshape(_FIELDS * _VOCAB, _EMB)

    info = plsc.get_sparse_core_info()
    nc, ns = info.num_cores, info.num_subcores
    nw = nc * ns
    rows_per_w = rows // nw
    chunk = 1024
    n_chunk = rows_per_w // chunk

    mesh = plsc.VectorSubcoreMesh(core_axis_name="c", subcore_axis_name="s")
    run = pl.kernel(
        functools.partial(_body, n_chunk, chunk, rows_per_w, nc),
        out_type=jax.ShapeDtypeStruct((rows, _EMB), jnp.float32),
        mesh=mesh,
        compiler_params=pltpu.CompilerParams(use_tc_tiling_on_sc=False),
        scratch_types=[
            pltpu.VMEM((chunk,), jnp.int32),
            pltpu.VMEM((chunk, _EMB), jnp.float32),
            pltpu.SemaphoreType.DMA,
        ],
    )
    out = run(x_flat, tab_flat)
    return out.reshape(batch, _FIELDS * _EMB)


# speedup vs baseline: 1.2172x; 1.0124x over previous
import functools

import jax
import jax.numpy as jnp
from jax import lax
from jax.experimental import pallas as pl
from jax.experimental.pallas import tpu as pltpu
from jax.experimental.pallas import tpu_sc as plsc

_FIELDS = 26
_VOCAB = 100000
_EMB = 32
_LANES = 16
_CHUNK = 1024
_NBUF = 2


def _body(n_chunk, rows_per_w, nc, x_hbm, tab_hbm, out_hbm,
          idx_v, rows_v, sems):
    wid = lax.axis_index("s") * nc + lax.axis_index("c")
    base = wid * rows_per_w
    lane = lax.iota(jnp.int32, _LANES)

    pltpu.sync_copy(x_hbm.at[pl.ds(base, rows_per_w)], idx_v)

    def add_offs(j, _):
        r = base + j * _LANES + lane
        f = lax.rem(r, _FIELDS)
        sl = pl.ds(j * _LANES, _LANES)
        idx_v[sl] = idx_v[sl] + f * _VOCAB
        return 0

    n_vec = rows_per_w // _LANES

    def gather(c, buf):
        return pltpu.async_copy(
            tab_hbm.at[idx_v.at[pl.ds(c * _CHUNK, _CHUNK)]],
            rows_v.at[buf], sems.at[buf])

    head_vec = (_NBUF * _CHUNK) // _LANES
    lax.fori_loop(0, head_vec, add_offs, 0, unroll=4)
    inflight = {b: gather(b, b) for b in range(_NBUF)}
    lax.fori_loop(head_vec, n_vec, add_offs, 0, unroll=4)

    for c in range(n_chunk):
        buf = c % _NBUF
        inflight.pop(c).wait()
        pltpu.sync_copy(rows_v.at[buf],
                        out_hbm.at[pl.ds(base + c * _CHUNK, _CHUNK)])
        nxt = c + _NBUF
        if nxt < n_chunk:
            inflight[nxt] = gather(nxt, buf)


def kernel(x, tables):
    batch = x.shape[0]
    rows = batch * _FIELDS
    x_flat = x.reshape(rows)
    tab_flat = tables.reshape(_FIELDS * _VOCAB, _EMB)

    info = plsc.get_sparse_core_info()
    nc, ns = info.num_cores, info.num_subcores
    nw = nc * ns
    rows_per_w = rows // nw
    n_chunk = rows_per_w // _CHUNK

    mesh = plsc.VectorSubcoreMesh(core_axis_name="c", subcore_axis_name="s")
    run = pl.kernel(
        functools.partial(_body, n_chunk, rows_per_w, nc),
        out_type=jax.ShapeDtypeStruct((rows, _EMB), jnp.float32),
        mesh=mesh,
        compiler_params=pltpu.CompilerParams(use_tc_tiling_on_sc=False),
        scratch_types=[
            pltpu.VMEM((rows_per_w,), jnp.int32),
            pltpu.VMEM((_NBUF, _CHUNK, _EMB), jnp.float32),
            pltpu.SemaphoreType.DMA((_NBUF,)),
        ],
    )
    out = run(x_flat, tab_flat)
    return out.reshape(batch, _FIELDS * _EMB)
